# baseline (speedup 1.0000x reference)
import jax
import jax.numpy as jnp
from jax import lax
from jax.experimental import pallas as pl
from jax.experimental.pallas import tpu as pltpu
from jax.experimental.pallas import tpu_sc as plsc

NUM_FIELDS = 26
CARD = 100
OUT_D = NUM_FIELDS * CARD
ROWS = 16384
NC, NS = 2, 16
NW = NC * NS
N_CT = OUT_D // 8
CT_STRIDE = (ROWS // 128) * 1024
BPW = ROWS // NW
SEG = (BPW // 128) * 1024
HALF = CT_STRIDE // NC
N_ENT = NUM_FIELDS * BPW
IDX_ROWS = N_ENT // 128


def _sc_body(xt_hbm, out_hbm, x_v, idx_v, zbuf, ones_v, sem):
    cid = lax.axis_index("c")
    sid = lax.axis_index("s")
    wid = cid * NS + sid
    b0 = wid * BPW

    zeros16 = jnp.zeros((16,), jnp.float32)
    ones16 = jnp.ones((16,), jnp.float32)
    iota16 = lax.iota(jnp.int32, 16)

    pltpu.sync_copy(xt_hbm.at[:, pl.ds(b0, BPW)], x_v)

    def zinit(j, carry):
        zbuf[pl.ds(pl.multiple_of(j * 16, 16), 16)] = zeros16
        return carry
    lax.fori_loop(0, HALF // 16, zinit, 0)

    def oinit(j, carry):
        ones_v[pl.ds(pl.multiple_of(j * 16, 16), 16)] = ones16
        return carry
    lax.fori_loop(0, N_ENT // 16, oinit, 0)

    def ient(e, carry):
        f = e // (BPW // 16)
        j = e % (BPW // 16)
        b_local = 16 * j + iota16
        xv = x_v[f, pl.ds(pl.multiple_of(16 * j, 16), 16)]
        c = xv + CARD * f
        idx = (
            ((c >> 3) << 17)
            + ((4 * wid + (b_local >> 7)) << 10)
            + ((c & 7) << 7)
            + (b_local & 127)
        )
        idx_v[pl.ds(pl.multiple_of(16 * e, 16), 16)] = idx
        return carry
    lax.fori_loop(0, N_ENT // 16, ient, 0)

    nct = (N_CT - sid + NS - 1) // NS

    def fill(k, carry):
        ct = sid + NS * k
        dst = out_hbm.at[pl.ds(ct * CT_STRIDE + cid * HALF, HALF)]
        pltpu.make_async_copy(zbuf, dst, sem).start()
        return carry
    lax.fori_loop(0, nct, fill, 0)

    def fill_wait(k, carry):
        ct = sid + NS * k
        dst = out_hbm.at[pl.ds(ct * CT_STRIDE + cid * HALF, HALF)]
        pltpu.make_async_copy(zbuf, dst, sem).wait()
        return carry
    lax.fori_loop(0, nct, fill_wait, 0)

    plsc.subcore_barrier()

    scat = pltpu.make_async_copy(ones_v, out_hbm.at[idx_v], sem)
    scat.start()
    scat.wait()


def kernel(x):
    xt = x.T
    mesh = plsc.VectorSubcoreMesh(core_axis_name="c", subcore_axis_name="s")
    f = pl.kernel(
        _sc_body,
        out_type=jax.ShapeDtypeStruct((OUT_D * ROWS,), jnp.float32),
        mesh=mesh,
        scratch_types=[
            pltpu.VMEM((NUM_FIELDS, BPW), jnp.int32),
            pltpu.VMEM((N_ENT,), jnp.int32),
            pltpu.VMEM((HALF,), jnp.float32),
            pltpu.VMEM((N_ENT,), jnp.float32),
            pltpu.SemaphoreType.DMA,
        ],
    )
    out1d = f(xt)
    out4 = out1d.reshape(N_CT, ROWS // 128, 8, 128)
    return out4.transpose(1, 3, 0, 2).reshape(ROWS, OUT_D)


# speedup vs baseline: 4.1182x; 4.1182x over previous
import jax
import jax.numpy as jnp
from jax import lax
from jax.experimental import pallas as pl
from jax.experimental.pallas import tpu as pltpu
from jax.experimental.pallas import tpu_sc as plsc

NUM_FIELDS = 26
CARD = 100
OUT_D = NUM_FIELDS * CARD
ROWS = 16384
NC, NS = 2, 16
NW = NC * NS
N_CT = OUT_D // 8
CT_STRIDE = (ROWS // 128) * 1024
BPW = ROWS // NW
SEG = (BPW // 128) * 1024
G = 13
NCHUNK = N_CT // G
CHW = G * SEG
JV = BPW // 16


def _sc_body(xt_hbm, out_hbm, x_v, buf0, buf1, sem0, sem1):
    cid = lax.axis_index("c")
    sid = lax.axis_index("s")
    wid = cid * NS + sid
    b0 = wid * BPW

    zeros16 = jnp.zeros((16,), jnp.float32)
    ones16 = jnp.ones((16,), jnp.float32)
    iota16 = lax.iota(jnp.int32, 16)

    pltpu.sync_copy(xt_hbm.at[:, pl.ds(b0, BPW)], x_v)

    def zinit(j, carry):
        s = pl.multiple_of(j * 16, 16)
        buf0[pl.ds(s, 16)] = zeros16
        buf1[pl.ds(s, 16)] = zeros16
        return carry
    lax.fori_loop(0, CHW // 16 + 1, zinit, 0)

    def scan_pass(buf, g, vals):
        f0 = (G * 8 * g) // CARD
        for df in range(3):
            f = jnp.minimum(f0 + df, NUM_FIELDS - 1)
            for j in range(JV):
                b_local = 16 * j + iota16
                xv = x_v[f, pl.ds(16 * j, 16)]
                c = xv + CARD * f
                lo = c - (G * 8) * g
                m = (lo >= 0) & (lo < G * 8)
                a = (
                    ((lo >> 3) << 12)
                    + ((b_local >> 7) << 10)
                    + ((lo & 7) << 7)
                    + (b_local & 127)
                )
                a = jnp.where(m, a, CHW + iota16)
                plsc.store_scatter(buf, [a], vals)

    def fire(buf, sem, g):
        for s in range(G):
            dst = out_hbm.at[pl.ds((G * g + s) * CT_STRIDE + wid * SEG, SEG)]
            pltpu.make_async_copy(buf.at[pl.ds(4096 * s, SEG)], dst, sem).start()

    def drain(buf, sem, g):
        for s in range(G):
            dst = out_hbm.at[pl.ds((G * g + s) * CT_STRIDE + wid * SEG, SEG)]
            pltpu.make_async_copy(buf.at[pl.ds(4096 * s, SEG)], dst, sem).wait()

    scan_pass(buf0, 0, ones16)
    fire(buf0, sem0, 0)

    def pair(p, carry):
        g1 = 2 * p + 1
        g2 = 2 * p + 2
        scan_pass(buf1, g1, ones16)
        fire(buf1, sem1, g1)
        drain(buf0, sem0, 2 * p)
        scan_pass(buf0, 2 * p, zeros16)
        scan_pass(buf0, g2, ones16)
        fire(buf0, sem0, g2)
        drain(buf1, sem1, g1)
        scan_pass(buf1, g1, zeros16)
        return carry
    lax.fori_loop(0, (NCHUNK - 1) // 2, pair, 0)

    drain(buf0, sem0, NCHUNK - 1)


def kernel(x):
    xt = x.T
    mesh = plsc.VectorSubcoreMesh(core_axis_name="c", subcore_axis_name="s")
    f = pl.kernel(
        _sc_body,
        out_type=jax.ShapeDtypeStruct((OUT_D * ROWS,), jnp.float32),
        mesh=mesh,
        compiler_params=pltpu.CompilerParams(needs_layout_passes=False),
        scratch_types=[
            pltpu.VMEM((NUM_FIELDS, BPW), jnp.int32),
            pltpu.VMEM((CHW + 16,), jnp.float32),
            pltpu.VMEM((CHW + 16,), jnp.float32),
            pltpu.SemaphoreType.DMA,
            pltpu.SemaphoreType.DMA,
        ],
    )
    out1d = f(xt)
    out4 = out1d.reshape(N_CT, ROWS // 128, 8, 128)
    return out4.transpose(1, 3, 0, 2).reshape(ROWS, OUT_D)


# speedup vs baseline: 5.3629x; 1.3022x over previous
import jax
import jax.numpy as jnp
from jax import lax
from jax.experimental import pallas as pl
from jax.experimental.pallas import tpu as pltpu
from jax.experimental.pallas import tpu_sc as plsc

NUM_FIELDS = 26
CARD = 100
OUT_D = NUM_FIELDS * CARD
ROWS = 16384
NC, NS = 2, 16
NW = NC * NS
N_CT = OUT_D // 8
CT_STRIDE = (ROWS // 128) * 1024
BPW = ROWS // NW
SEG = (BPW // 128) * 1024
G = 13
NCHUNK = N_CT // G
CHW = G * SEG
JV = BPW // 16


def _sc_body(xt_hbm, out_hbm, x_v, buf0, buf1, sem0, sem1):
    cid = lax.axis_index("c")
    sid = lax.axis_index("s")
    wid = cid * NS + sid
    b0 = wid * BPW

    zeros16 = jnp.zeros((16,), jnp.float32)
    ones16 = jnp.ones((16,), jnp.float32)
    iota16 = lax.iota(jnp.int32, 16)

    pltpu.sync_copy(xt_hbm.at[:, pl.ds(b0, BPW)], x_v)

    def precomp(e, carry):
        f = e // JV
        col = pl.multiple_of(16 * (e % JV), 16)
        b_local = 16 * (e % JV) + iota16
        c = x_v[f, pl.ds(col, 16)] + CARD * f
        x_v[f, pl.ds(col, 16)] = (
            ((c >> 3) << 12)
            + ((b_local >> 7) << 10)
            + ((c & 7) << 7)
            + (b_local & 127)
        )
        return carry
    lax.fori_loop(0, NUM_FIELDS * JV, precomp, 0)

    def zinit(j, carry):
        s = pl.multiple_of(j * 16, 16)
        buf0[pl.ds(s, 16)] = zeros16
        buf1[pl.ds(s, 16)] = zeros16
        return carry
    lax.fori_loop(0, CHW // 16 + 1, zinit, 0)

    def scan_pass(buf, g, vals):
        f0 = (G * 8 * g) // CARD
        for df in range(2):
            f = f0 + df
            for j in range(JV):
                a = x_v[f, pl.ds(16 * j, 16)] - g * CHW
                m = (a >= 0) & (a < CHW)
                a = jnp.where(m, a, CHW + iota16)
                plsc.store_scatter(buf, [a], vals)

    def fire(buf, sem, g):
        for s in range(G):
            dst = out_hbm.at[pl.ds((G * g + s) * CT_STRIDE + wid * SEG, SEG)]
            pltpu.make_async_copy(buf.at[pl.ds(4096 * s, SEG)], dst, sem).start()

    def drain(buf, sem, g):
        for s in range(G):
            dst = out_hbm.at[pl.ds((G * g + s) * CT_STRIDE + wid * SEG, SEG)]
            pltpu.make_async_copy(buf.at[pl.ds(4096 * s, SEG)], dst, sem).wait()

    scan_pass(buf0, 0, ones16)
    fire(buf0, sem0, 0)

    def pair(p, carry):
        g1 = 2 * p + 1
        g2 = 2 * p + 2
        scan_pass(buf1, g1, ones16)
        fire(buf1, sem1, g1)
        drain(buf0, sem0, 2 * p)
        scan_pass(buf0, 2 * p, zeros16)
        scan_pass(buf0, g2, ones16)
        fire(buf0, sem0, g2)
        drain(buf1, sem1, g1)
        scan_pass(buf1, g1, zeros16)
        return carry
    lax.fori_loop(0, (NCHUNK - 1) // 2, pair, 0)

    drain(buf0, sem0, NCHUNK - 1)


def kernel(x):
    xt = x.T
    mesh = plsc.VectorSubcoreMesh(core_axis_name="c", subcore_axis_name="s")
    f = pl.kernel(
        _sc_body,
        out_type=jax.ShapeDtypeStruct((OUT_D * ROWS,), jnp.float32),
        mesh=mesh,
        compiler_params=pltpu.CompilerParams(needs_layout_passes=False),
        scratch_types=[
            pltpu.VMEM((NUM_FIELDS, BPW), jnp.int32),
            pltpu.VMEM((CHW + 16,), jnp.float32),
            pltpu.VMEM((CHW + 16,), jnp.float32),
            pltpu.SemaphoreType.DMA,
            pltpu.SemaphoreType.DMA,
        ],
    )
    out1d = f(xt)
    out4 = out1d.reshape(N_CT, ROWS // 128, 8, 128)
    return out4.transpose(1, 3, 0, 2).reshape(ROWS, OUT_D)
